# baseline (device time: 16390 ns/iter reference)
import jax
import jax.numpy as jnp
from jax import lax
from jax.experimental import pallas as pl
from jax.experimental.pallas import tpu as pltpu

M = 1024
NCOL = 512
HALF = 512
C = 8
CH = HALF // C


def kernel(x):
    def body(
        x_ref, out_ref,
        xo_f32, xm_f32, other_buf, a_recv, red_buf,
        cp_sems, sa, ra, sb, rb, wsems,
    ):
        my_x = lax.axis_index("x")
        my_y = lax.axis_index("y")

        row0 = my_y * HALF
        col_mine = my_x * NCOL
        col_other = (1 - my_x) * NCOL

        cp_other = pltpu.make_async_copy(
            x_ref.at[0, pl.ds(row0, HALF), pl.ds(col_other, NCOL)],
            xo_f32, cp_sems.at[0],
        )
        cp_other.start()
        cp_mine = pltpu.make_async_copy(
            x_ref.at[0, pl.ds(row0, HALF), pl.ds(col_mine, NCOL)],
            xm_f32, cp_sems.at[1],
        )
        cp_mine.start()

        barrier = pltpu.get_barrier_semaphore()
        pl.semaphore_signal(
            barrier, inc=1, device_id=(1 - my_x, my_y),
            device_id_type=pl.DeviceIdType.MESH,
        )
        pl.semaphore_signal(
            barrier, inc=1, device_id=(my_x, 1 - my_y),
            device_id_type=pl.DeviceIdType.MESH,
        )

        cp_other.wait()
        other_buf[...] = xo_f32[...].astype(jnp.bfloat16)

        pl.semaphore_wait(barrier, 2)

        a_descs = []
        for c in range(C):
            d = pltpu.make_async_remote_copy(
                src_ref=other_buf.at[pl.ds(c * CH, CH), :],
                dst_ref=a_recv.at[pl.ds(c * CH, CH), :],
                send_sem=sa.at[c],
                recv_sem=ra.at[c],
                device_id=(1 - my_x, my_y),
                device_id_type=pl.DeviceIdType.MESH,
            )
            d.start()
            a_descs.append(d)

        cp_mine.wait()

        b_descs = []
        w_descs = []
        for c in range(C):
            a_descs[c].wait_recv()
            red_buf[pl.ds(c * CH, CH), :] = (
                xm_f32[pl.ds(c * CH, CH), :].astype(jnp.bfloat16)
                + a_recv[pl.ds(c * CH, CH), :]
            )
            w = pltpu.make_async_copy(
                red_buf.at[pl.ds(c * CH, CH), :],
                out_ref.at[pl.ds(row0 + c * CH, CH), :],
                wsems.at[c],
            )
            w.start()
            w_descs.append(w)
            d = pltpu.make_async_remote_copy(
                src_ref=red_buf.at[pl.ds(c * CH, CH), :],
                dst_ref=out_ref.at[pl.ds(row0 + c * CH, CH), :],
                send_sem=sb.at[c],
                recv_sem=rb.at[c],
                device_id=(my_x, 1 - my_y),
                device_id_type=pl.DeviceIdType.MESH,
            )
            d.start()
            b_descs.append(d)

        for c in range(C):
            a_descs[c].wait_send()
            w_descs[c].wait()
            b_descs[c].wait_send()
            b_descs[c].wait_recv()

    return pl.pallas_call(
        body,
        out_shape=jax.ShapeDtypeStruct((M, NCOL), jnp.bfloat16),
        in_specs=[pl.BlockSpec(memory_space=pl.ANY)],
        out_specs=pl.BlockSpec(memory_space=pl.ANY),
        scratch_shapes=[
            pltpu.VMEM((HALF, NCOL), jnp.float32),
            pltpu.VMEM((HALF, NCOL), jnp.float32),
            pltpu.VMEM((HALF, NCOL), jnp.bfloat16),
            pltpu.VMEM((HALF, NCOL), jnp.bfloat16),
            pltpu.VMEM((HALF, NCOL), jnp.bfloat16),
            pltpu.SemaphoreType.DMA((2,)),
            pltpu.SemaphoreType.DMA((C,)),
            pltpu.SemaphoreType.DMA((C,)),
            pltpu.SemaphoreType.DMA((C,)),
            pltpu.SemaphoreType.DMA((C,)),
            pltpu.SemaphoreType.DMA((C,)),
        ],
        compiler_params=pltpu.CompilerParams(collective_id=0),
    )(x)


# device time: 16023 ns/iter; 1.0229x vs baseline; 1.0229x over previous
import jax
import jax.numpy as jnp
from jax import lax
from jax.experimental import pallas as pl
from jax.experimental.pallas import tpu as pltpu

M = 1024
NCOL = 512
HALF = 512
C = 8
CH = HALF // C


def kernel(x):
    def body(
        x_ref, out_ref,
        xo_f32, xm_f32, other_buf, a_recv,
        cp_sems, sa, ra, sb, rb,
    ):
        my_x = lax.axis_index("x")
        my_y = lax.axis_index("y")

        row0 = my_y * HALF
        col_mine = my_x * NCOL
        col_other = (1 - my_x) * NCOL

        cp_other = pltpu.make_async_copy(
            x_ref.at[0, pl.ds(row0, HALF), pl.ds(col_other, NCOL)],
            xo_f32, cp_sems.at[0],
        )
        cp_other.start()
        cp_mine = pltpu.make_async_copy(
            x_ref.at[0, pl.ds(row0, HALF), pl.ds(col_mine, NCOL)],
            xm_f32, cp_sems.at[1],
        )
        cp_mine.start()

        barrier = pltpu.get_barrier_semaphore()
        pl.semaphore_signal(
            barrier, inc=1, device_id=(1 - my_x, my_y),
            device_id_type=pl.DeviceIdType.MESH,
        )
        pl.semaphore_signal(
            barrier, inc=1, device_id=(my_x, 1 - my_y),
            device_id_type=pl.DeviceIdType.MESH,
        )

        cp_other.wait()
        other_buf[...] = xo_f32[...].astype(jnp.bfloat16)

        pl.semaphore_wait(barrier, 2)

        a_descs = []
        for c in range(C):
            d = pltpu.make_async_remote_copy(
                src_ref=other_buf.at[pl.ds(c * CH, CH), :],
                dst_ref=a_recv.at[pl.ds(c * CH, CH), :],
                send_sem=sa.at[c],
                recv_sem=ra.at[c],
                device_id=(1 - my_x, my_y),
                device_id_type=pl.DeviceIdType.MESH,
            )
            d.start()
            a_descs.append(d)

        cp_mine.wait()

        b_descs = []
        for c in range(C):
            a_descs[c].wait_recv()
            out_ref[pl.ds(row0 + c * CH, CH), :] = (
                xm_f32[pl.ds(c * CH, CH), :].astype(jnp.bfloat16)
                + a_recv[pl.ds(c * CH, CH), :]
            )
            d = pltpu.make_async_remote_copy(
                src_ref=out_ref.at[pl.ds(row0 + c * CH, CH), :],
                dst_ref=out_ref.at[pl.ds(row0 + c * CH, CH), :],
                send_sem=sb.at[c],
                recv_sem=rb.at[c],
                device_id=(my_x, 1 - my_y),
                device_id_type=pl.DeviceIdType.MESH,
            )
            d.start()
            b_descs.append(d)

        for c in range(C):
            a_descs[c].wait_send()
            b_descs[c].wait_send()
            b_descs[c].wait_recv()

    return pl.pallas_call(
        body,
        out_shape=jax.ShapeDtypeStruct((M, NCOL), jnp.bfloat16),
        in_specs=[pl.BlockSpec(memory_space=pl.ANY)],
        out_specs=pl.BlockSpec(memory_space=pltpu.VMEM),
        scratch_shapes=[
            pltpu.VMEM((HALF, NCOL), jnp.float32),
            pltpu.VMEM((HALF, NCOL), jnp.float32),
            pltpu.VMEM((HALF, NCOL), jnp.bfloat16),
            pltpu.VMEM((HALF, NCOL), jnp.bfloat16),
            pltpu.SemaphoreType.DMA((2,)),
            pltpu.SemaphoreType.DMA((C,)),
            pltpu.SemaphoreType.DMA((C,)),
            pltpu.SemaphoreType.DMA((C,)),
            pltpu.SemaphoreType.DMA((C,)),
        ],
        compiler_params=pltpu.CompilerParams(collective_id=0),
    )(x)
